# baseline (speedup 1.0000x reference)
import functools

import jax
import jax.numpy as jnp
from jax import lax
from jax.experimental import pallas as pl
from jax.experimental.pallas import tpu as pltpu
from jax.experimental.pallas import tpu_sc as plsc

N = 2048
K = 8
OUT_PER = 8
BR = 256
NC = 2
NS = 16
NW = NC * NS
TC_ROWS = 1280
SC_ROWS = N - TC_ROWS
SC_RPW = SC_ROWS // NW
CHUNKS = N // 16
INF = float("inf")


def _sc_body(x1h, y1h, x2h, y2h, wth, bth, outh,
             x1v, y1v, x2v, y2v, vxv, vyv, wtv, btv, fbuf, outv,
             dbuf, ckbuf, cvbuf, tbuf):
    wid = lax.axis_index("s") * NC + lax.axis_index("c")
    base_row = TC_ROWS + wid * SC_RPW

    pltpu.sync_copy(x1h, x1v)
    pltpu.sync_copy(y1h, y1v)
    pltpu.sync_copy(x2h, x2v)
    pltpu.sync_copy(y2h, y2v)
    pltpu.sync_copy(wth, wtv)
    pltpu.sync_copy(bth, btv)

    io = lax.iota(jnp.int32, 16)
    khalf = lax.shift_right_logical(io, 3)

    def _vel(c, carry):
        s = pl.ds(c * 16, 16)
        vxv[s] = x2v[s] - x1v[s]
        vyv[s] = y2v[s] - y1v[s]
        return carry
    lax.fori_loop(0, CHUNKS, _vel, 0)

    btile = btv[...]
    w0 = wtv[0, :]
    w1 = wtv[1, :]
    w2 = wtv[2, :]
    w3 = wtv[3, :]

    def _agent(a, carry):
        i = base_row + a
        ivec = jnp.full((16,), i, jnp.int32)
        xi = plsc.load_gather(x2v, [ivec])
        yi = plsc.load_gather(y2v, [ivec])
        vxi = plsc.load_gather(vxv, [ivec])
        vyi = plsc.load_gather(vyv, [ivec])

        def _passa(c, rmin):
            s = pl.ds(c * 16, 16)
            civ = io + c * 16
            dx = x2v[s] - xi
            dy = y2v[s] - yi
            d = dx * dx + dy * dy
            d = jnp.where(civ == ivec, INF, d)
            dbuf[s] = d
            return jnp.minimum(rmin, d)

        rmin = lax.fori_loop(0, CHUNKS, _passa,
                             jnp.full((16,), INF, jnp.float32))

        srt, _ = plsc.sort_key_val(rmin, io)
        tbuf[pl.ds(0, 16)] = srt
        tau = plsc.load_gather(tbuf, [jnp.full((16,), 7, jnp.int32)])

        def _passb(c, off):
            s = pl.ds(c * 16, 16)
            d = dbuf[s]
            civ = io + c * 16
            m = d <= tau
            plsc.store_compressed(ckbuf.at[pl.ds(off, 16)], d, mask=m)
            plsc.store_compressed(cvbuf.at[pl.ds(off, 16)], civ, mask=m)
            cnt = plsc.all_reduce_population_count(m)
            return off + cnt[0]

        ncand = lax.fori_loop(0, CHUNKS, _passb, jnp.int32(0))
        ckbuf[pl.ds(ncand, 16)] = jnp.full((16,), INF, jnp.float32)

        def _merge(t, bkv):
            bk, bvv = bkv
            s = pl.ds(t * 16, 16)
            cks, cvs = plsc.sort_key_val(ckbuf[s], cvbuf[s])
            rk = lax.rev(cks, (0,))
            rv = lax.rev(cvs, (0,))
            m = bk <= rk
            lk = jnp.where(m, bk, rk)
            lv = jnp.where(m, bvv, rv)
            nk, nv = plsc.sort_key_val(lk, lv)
            return (nk, nv)

        bk0 = jnp.full((16,), INF, jnp.float32)
        bv0 = jnp.zeros((16,), jnp.int32)
        _, bv = lax.fori_loop(0, (ncand + 15) // 16, _merge, (bk0, bv0))

        gx = plsc.load_gather(x2v, [bv])
        gy = plsc.load_gather(y2v, [bv])
        gvx = plsc.load_gather(vxv, [bv])
        gvy = plsc.load_gather(vyv, [bv])
        fbuf[pl.ds(0, 16)] = gx - xi
        fbuf[pl.ds(16, 16)] = gy - yi
        fbuf[pl.ds(32, 16)] = gvx - vxi
        fbuf[pl.ds(48, 16)] = gvy - vyi

        for g in range(4):
            sel = khalf + (2 * g)
            acc = btile
            acc = acc + plsc.load_gather(fbuf, [sel]) * w0
            acc = acc + plsc.load_gather(fbuf, [sel + 16]) * w1
            acc = acc + plsc.load_gather(fbuf, [sel + 32]) * w2
            acc = acc + plsc.load_gather(fbuf, [sel + 48]) * w3
            outv[a, pl.ds(g * 16, 16)] = jnp.maximum(acc, 0.0)
        return carry

    lax.fori_loop(0, SC_RPW, _agent, 0)

    pltpu.sync_copy(outv, outh.at[pl.ds(wid * SC_RPW, SC_RPW)])


def _sc_run(x1, y1, x2, y2, W, b):
    wt = jnp.tile(W.T, (1, 2))
    bt = jnp.tile(b, 2)
    mesh = plsc.VectorSubcoreMesh(
        core_axis_name="c", subcore_axis_name="s",
        num_cores=NC, num_subcores=NS)
    kern = functools.partial(
        pl.kernel,
        out_type=jax.ShapeDtypeStruct((SC_ROWS, K * OUT_PER), jnp.float32),
        mesh=mesh,
        compiler_params=pltpu.CompilerParams(
            use_tc_tiling_on_sc=False, needs_layout_passes=False),
        scratch_types=[
            pltpu.VMEM((N,), jnp.float32),
            pltpu.VMEM((N,), jnp.float32),
            pltpu.VMEM((N,), jnp.float32),
            pltpu.VMEM((N,), jnp.float32),
            pltpu.VMEM((N,), jnp.float32),
            pltpu.VMEM((N,), jnp.float32),
            pltpu.VMEM((4, 16), jnp.float32),
            pltpu.VMEM((16,), jnp.float32),
            pltpu.VMEM((64,), jnp.float32),
            pltpu.VMEM((SC_RPW, K * OUT_PER), jnp.float32),
            pltpu.VMEM((N,), jnp.float32),
            pltpu.VMEM((N + 16,), jnp.float32),
            pltpu.VMEM((N + 16,), jnp.int32),
            pltpu.VMEM((16,), jnp.float32),
        ],
    )(_sc_body)
    return kern(x1, y1, x2, y2, wt, bt)


def _tc_body(x1r, y1r, x2c, y2c, x2r, y2r, wt, b2, out_ref):
    i = pl.program_id(0)
    base = i * BR

    col = lax.broadcasted_iota(jnp.int32, (BR, N), 1)
    row = base + lax.broadcasted_iota(jnp.int32, (BR, N), 0)

    relx = x2r[...] - x2c[...]
    rely = y2r[...] - y2c[...]
    dist = jnp.sqrt(relx * relx + rely * rely)
    dist = jnp.where(col == row, jnp.inf, dist)

    vxr = x2r[...] - x1r[...]
    vyr = y2r[...] - y1r[...]
    ptab = jnp.concatenate([x2r[...], y2r[...], vxr, vyr], axis=0).T

    rowhot = (col == row).astype(jnp.float32)
    self4 = jnp.dot(rowhot, ptab, preferred_element_type=jnp.float32)

    for k in range(K):
        m = jnp.min(dist, axis=1, keepdims=True)
        cand = jnp.where(dist == m, col, N)
        idx = jnp.min(cand, axis=1, keepdims=True)
        onehot = (col == idx).astype(jnp.float32)
        feats = jnp.dot(onehot, ptab, preferred_element_type=jnp.float32)
        rel = feats - self4
        emb = jnp.maximum(
            jnp.dot(rel, wt[...], preferred_element_type=jnp.float32)
            + b2[...], 0.0)
        out_ref[:, k * OUT_PER:(k + 1) * OUT_PER] = emb
        if k != K - 1:
            dist = jnp.where(col == idx, jnp.inf, dist)


def _tc_run(x1, y1, x2, y2, W, b):
    x1r = x1.reshape(1, N)
    y1r = y1.reshape(1, N)
    x2r = x2.reshape(1, N)
    y2r = y2.reshape(1, N)
    x2c = x2.reshape(N, 1)
    y2c = y2.reshape(N, 1)
    wt = W.T
    b2 = b.reshape(1, OUT_PER)

    grid = (TC_ROWS // BR,)
    full_row = pl.BlockSpec((1, N), lambda i: (0, 0))
    col_blk = pl.BlockSpec((BR, 1), lambda i: (i, 0))
    return pl.pallas_call(
        _tc_body,
        grid=grid,
        in_specs=[
            full_row, full_row,
            col_blk, col_blk,
            full_row, full_row,
            pl.BlockSpec((4, OUT_PER), lambda i: (0, 0)),
            pl.BlockSpec((1, OUT_PER), lambda i: (0, 0)),
        ],
        out_specs=pl.BlockSpec((BR, K * OUT_PER), lambda i: (i, 0)),
        out_shape=jax.ShapeDtypeStruct((TC_ROWS, K * OUT_PER), jnp.float32),
    )(x1r, y1r, x2c, y2c, x2r, y2r, wt, b2)


@jax.jit
def _run(obs1, obs2, W, b):
    x1 = obs1[:, 0]
    y1 = obs1[:, 1]
    x2 = obs2[:, 0]
    y2 = obs2[:, 1]
    sc_out = _sc_run(x1, y1, x2, y2, W, b)
    tc_out = _tc_run(x1, y1, x2, y2, W, b)
    return jnp.concatenate([tc_out, sc_out], axis=0)


def kernel(_, obs1, obs2, W, b):
    return _run(obs1, obs2, W, b)


# speedup vs baseline: 1.3107x; 1.3107x over previous
import functools

import jax
import jax.numpy as jnp
from jax import lax
from jax.experimental import pallas as pl
from jax.experimental.pallas import tpu as pltpu
from jax.experimental.pallas import tpu_sc as plsc

N = 2048
K = 8
OUT_PER = 8
BR = 256
NC = 2
NS = 16
NW = NC * NS
TC_ROWS = 1280
SC_ROWS = N - TC_ROWS
SC_RPW = SC_ROWS // NW
CHUNKS = N // 16
INF = float("inf")


def _sc_body(x1h, y1h, x2h, y2h, wth, bth, outh,
             x1v, y1v, x2v, y2v, vxv, vyv, wtv, btv, fbuf, outv):
    wid = lax.axis_index("s") * NC + lax.axis_index("c")
    base_row = TC_ROWS + wid * SC_RPW

    pltpu.sync_copy(x1h, x1v)
    pltpu.sync_copy(y1h, y1v)
    pltpu.sync_copy(x2h, x2v)
    pltpu.sync_copy(y2h, y2v)
    pltpu.sync_copy(wth, wtv)
    pltpu.sync_copy(bth, btv)

    io = lax.iota(jnp.int32, 16)
    khalf = lax.shift_right_logical(io, 3)

    def _vel(c, carry):
        s = pl.ds(c * 16, 16)
        vxv[s] = x2v[s] - x1v[s]
        vyv[s] = y2v[s] - y1v[s]
        return carry
    lax.fori_loop(0, CHUNKS, _vel, 0)

    btile = btv[...]
    w0 = wtv[0, :]
    w1 = wtv[1, :]
    w2 = wtv[2, :]
    w3 = wtv[3, :]

    def _post(a, xi, yi, vxi, vyi, bv):
        gx = plsc.load_gather(x2v, [bv])
        gy = plsc.load_gather(y2v, [bv])
        gvx = plsc.load_gather(vxv, [bv])
        gvy = plsc.load_gather(vyv, [bv])
        fbuf[pl.ds(0, 16)] = gx - xi
        fbuf[pl.ds(16, 16)] = gy - yi
        fbuf[pl.ds(32, 16)] = gvx - vxi
        fbuf[pl.ds(48, 16)] = gvy - vyi

        for g in range(4):
            sel = khalf + (2 * g)
            acc = btile
            acc = acc + plsc.load_gather(fbuf, [sel]) * w0
            acc = acc + plsc.load_gather(fbuf, [sel + 16]) * w1
            acc = acc + plsc.load_gather(fbuf, [sel + 32]) * w2
            acc = acc + plsc.load_gather(fbuf, [sel + 48]) * w3
            outv[a, pl.ds(g * 16, 16)] = jnp.maximum(acc, 0.0)

    def _agent_pair(p, carry):
        a0 = p * 2
        a1 = a0 + 1
        iv0 = jnp.full((16,), base_row + a0, jnp.int32)
        iv1 = jnp.full((16,), base_row + a1, jnp.int32)
        xi0 = plsc.load_gather(x2v, [iv0])
        yi0 = plsc.load_gather(y2v, [iv0])
        vxi0 = plsc.load_gather(vxv, [iv0])
        vyi0 = plsc.load_gather(vyv, [iv0])
        xi1 = plsc.load_gather(x2v, [iv1])
        yi1 = plsc.load_gather(y2v, [iv1])
        vxi1 = plsc.load_gather(vxv, [iv1])
        vyi1 = plsc.load_gather(vyv, [iv1])

        def _chunk(c, bkv):
            bk0, bv0, bk1, bv1 = bkv
            s = pl.ds(c * 16, 16)
            civ = io + c * 16
            xs = x2v[s]
            ys = y2v[s]

            dx0 = xs - xi0
            dy0 = ys - yi0
            d0 = dx0 * dx0 + dy0 * dy0
            d0 = jnp.where(civ == iv0, INF, d0)
            ck0, cw0 = plsc.sort_key_val(d0, civ)

            dx1 = xs - xi1
            dy1 = ys - yi1
            d1 = dx1 * dx1 + dy1 * dy1
            d1 = jnp.where(civ == iv1, INF, d1)
            ck1, cw1 = plsc.sort_key_val(d1, civ)

            rk0 = lax.rev(ck0, (0,))
            rv0 = lax.rev(cw0, (0,))
            m0 = bk0 <= rk0
            nk0, nv0 = plsc.sort_key_val(
                jnp.where(m0, bk0, rk0), jnp.where(m0, bv0, rv0))

            rk1 = lax.rev(ck1, (0,))
            rv1 = lax.rev(cw1, (0,))
            m1 = bk1 <= rk1
            nk1, nv1 = plsc.sort_key_val(
                jnp.where(m1, bk1, rk1), jnp.where(m1, bv1, rv1))
            return (nk0, nv0, nk1, nv1)

        inf16 = jnp.full((16,), INF, jnp.float32)
        z16 = jnp.zeros((16,), jnp.int32)
        _, bva, _, bvb = lax.fori_loop(
            0, CHUNKS, _chunk, (inf16, z16, inf16, z16))

        _post(a0, xi0, yi0, vxi0, vyi0, bva)
        _post(a1, xi1, yi1, vxi1, vyi1, bvb)
        return carry

    lax.fori_loop(0, SC_RPW // 2, _agent_pair, 0)

    pltpu.sync_copy(outv, outh.at[pl.ds(wid * SC_RPW, SC_RPW)])


def _sc_run(x1, y1, x2, y2, W, b):
    wt = jnp.tile(W.T, (1, 2))
    bt = jnp.tile(b, 2)
    mesh = plsc.VectorSubcoreMesh(
        core_axis_name="c", subcore_axis_name="s",
        num_cores=NC, num_subcores=NS)
    kern = functools.partial(
        pl.kernel,
        out_type=jax.ShapeDtypeStruct((SC_ROWS, K * OUT_PER), jnp.float32),
        mesh=mesh,
        compiler_params=pltpu.CompilerParams(
            use_tc_tiling_on_sc=False, needs_layout_passes=False),
        scratch_types=[
            pltpu.VMEM((N,), jnp.float32),
            pltpu.VMEM((N,), jnp.float32),
            pltpu.VMEM((N,), jnp.float32),
            pltpu.VMEM((N,), jnp.float32),
            pltpu.VMEM((N,), jnp.float32),
            pltpu.VMEM((N,), jnp.float32),
            pltpu.VMEM((4, 16), jnp.float32),
            pltpu.VMEM((16,), jnp.float32),
            pltpu.VMEM((64,), jnp.float32),
            pltpu.VMEM((SC_RPW, K * OUT_PER), jnp.float32),
        ],
    )(_sc_body)
    return kern(x1, y1, x2, y2, wt, bt)


def _tc_body(x1r, y1r, x2c, y2c, x2r, y2r, wt, b2, out_ref):
    i = pl.program_id(0)
    base = i * BR

    col = lax.broadcasted_iota(jnp.int32, (BR, N), 1)
    row = base + lax.broadcasted_iota(jnp.int32, (BR, N), 0)

    relx = x2r[...] - x2c[...]
    rely = y2r[...] - y2c[...]
    dist = jnp.sqrt(relx * relx + rely * rely)
    dist = jnp.where(col == row, jnp.inf, dist)

    vxr = x2r[...] - x1r[...]
    vyr = y2r[...] - y1r[...]
    ptab = jnp.concatenate([x2r[...], y2r[...], vxr, vyr], axis=0).T

    rowhot = (col == row).astype(jnp.float32)
    self4 = jnp.dot(rowhot, ptab, preferred_element_type=jnp.float32)

    for k in range(K):
        m = jnp.min(dist, axis=1, keepdims=True)
        cand = jnp.where(dist == m, col, N)
        idx = jnp.min(cand, axis=1, keepdims=True)
        onehot = (col == idx).astype(jnp.float32)
        feats = jnp.dot(onehot, ptab, preferred_element_type=jnp.float32)
        rel = feats - self4
        emb = jnp.maximum(
            jnp.dot(rel, wt[...], preferred_element_type=jnp.float32)
            + b2[...], 0.0)
        out_ref[:, k * OUT_PER:(k + 1) * OUT_PER] = emb
        if k != K - 1:
            dist = jnp.where(col == idx, jnp.inf, dist)


def _tc_run(x1, y1, x2, y2, W, b):
    x1r = x1.reshape(1, N)
    y1r = y1.reshape(1, N)
    x2r = x2.reshape(1, N)
    y2r = y2.reshape(1, N)
    x2c = x2.reshape(N, 1)
    y2c = y2.reshape(N, 1)
    wt = W.T
    b2 = b.reshape(1, OUT_PER)

    grid = (TC_ROWS // BR,)
    full_row = pl.BlockSpec((1, N), lambda i: (0, 0))
    col_blk = pl.BlockSpec((BR, 1), lambda i: (i, 0))
    return pl.pallas_call(
        _tc_body,
        grid=grid,
        in_specs=[
            full_row, full_row,
            col_blk, col_blk,
            full_row, full_row,
            pl.BlockSpec((4, OUT_PER), lambda i: (0, 0)),
            pl.BlockSpec((1, OUT_PER), lambda i: (0, 0)),
        ],
        out_specs=pl.BlockSpec((BR, K * OUT_PER), lambda i: (i, 0)),
        out_shape=jax.ShapeDtypeStruct((TC_ROWS, K * OUT_PER), jnp.float32),
    )(x1r, y1r, x2c, y2c, x2r, y2r, wt, b2)


@jax.jit
def _run(obs1, obs2, W, b):
    x1 = obs1[:, 0]
    y1 = obs1[:, 1]
    x2 = obs2[:, 0]
    y2 = obs2[:, 1]
    sc_out = _sc_run(x1, y1, x2, y2, W, b)
    tc_out = _tc_run(x1, y1, x2, y2, W, b)
    return jnp.concatenate([tc_out, sc_out], axis=0)


def kernel(_, obs1, obs2, W, b):
    return _run(obs1, obs2, W, b)


# speedup vs baseline: 1.5156x; 1.1563x over previous
import functools

import jax
import jax.numpy as jnp
from jax import lax
from jax.experimental import pallas as pl
from jax.experimental.pallas import tpu as pltpu
from jax.experimental.pallas import tpu_sc as plsc

N = 2048
K = 8
OUT_PER = 8
BR = 256
NC = 2
NS = 16
NW = NC * NS
TC_ROWS = 1024
SC_ROWS = N - TC_ROWS
SC_RPW = SC_ROWS // NW
CHUNKS = N // 16
INF = float("inf")


def _sc_body(x1h, y1h, x2h, y2h, wth, bth, outh,
             x1v, y1v, x2v, y2v, vxv, vyv, wtv, btv, fbuf, outv):
    wid = lax.axis_index("s") * NC + lax.axis_index("c")
    base_row = TC_ROWS + wid * SC_RPW

    pltpu.sync_copy(x1h, x1v)
    pltpu.sync_copy(y1h, y1v)
    pltpu.sync_copy(x2h, x2v)
    pltpu.sync_copy(y2h, y2v)
    pltpu.sync_copy(wth, wtv)
    pltpu.sync_copy(bth, btv)

    io = lax.iota(jnp.int32, 16)
    khalf = lax.shift_right_logical(io, 3)

    def _vel(c, carry):
        s = pl.ds(c * 16, 16)
        vxv[s] = x2v[s] - x1v[s]
        vyv[s] = y2v[s] - y1v[s]
        return carry
    lax.fori_loop(0, CHUNKS, _vel, 0)

    btile = btv[...]
    w0 = wtv[0, :]
    w1 = wtv[1, :]
    w2 = wtv[2, :]
    w3 = wtv[3, :]

    def _post(a, xi, yi, vxi, vyi, bv):
        gx = plsc.load_gather(x2v, [bv])
        gy = plsc.load_gather(y2v, [bv])
        gvx = plsc.load_gather(vxv, [bv])
        gvy = plsc.load_gather(vyv, [bv])
        fbuf[pl.ds(0, 16)] = gx - xi
        fbuf[pl.ds(16, 16)] = gy - yi
        fbuf[pl.ds(32, 16)] = gvx - vxi
        fbuf[pl.ds(48, 16)] = gvy - vyi

        for g in range(4):
            sel = khalf + (2 * g)
            acc = btile
            acc = acc + plsc.load_gather(fbuf, [sel]) * w0
            acc = acc + plsc.load_gather(fbuf, [sel + 16]) * w1
            acc = acc + plsc.load_gather(fbuf, [sel + 32]) * w2
            acc = acc + plsc.load_gather(fbuf, [sel + 48]) * w3
            outv[a, pl.ds(g * 16, 16)] = jnp.maximum(acc, 0.0)

    def _agent_pair(p, carry):
        a0 = p * 2
        a1 = a0 + 1
        iv0 = jnp.full((16,), base_row + a0, jnp.int32)
        iv1 = jnp.full((16,), base_row + a1, jnp.int32)
        xi0 = plsc.load_gather(x2v, [iv0])
        yi0 = plsc.load_gather(y2v, [iv0])
        vxi0 = plsc.load_gather(vxv, [iv0])
        vyi0 = plsc.load_gather(vyv, [iv0])
        xi1 = plsc.load_gather(x2v, [iv1])
        yi1 = plsc.load_gather(y2v, [iv1])
        vxi1 = plsc.load_gather(vxv, [iv1])
        vyi1 = plsc.load_gather(vyv, [iv1])

        def _chunk(c, bkv):
            bk0, bv0, bk1, bv1 = bkv
            s = pl.ds(c * 16, 16)
            civ = io + c * 16
            xs = x2v[s]
            ys = y2v[s]

            dx0 = xs - xi0
            dy0 = ys - yi0
            d0 = dx0 * dx0 + dy0 * dy0
            d0 = jnp.where(civ == iv0, INF, d0)
            ck0, cw0 = plsc.sort_key_val(d0, civ)

            dx1 = xs - xi1
            dy1 = ys - yi1
            d1 = dx1 * dx1 + dy1 * dy1
            d1 = jnp.where(civ == iv1, INF, d1)
            ck1, cw1 = plsc.sort_key_val(d1, civ)

            rk0 = lax.rev(ck0, (0,))
            rv0 = lax.rev(cw0, (0,))
            m0 = bk0 <= rk0
            nk0, nv0 = plsc.sort_key_val(
                jnp.where(m0, bk0, rk0), jnp.where(m0, bv0, rv0))

            rk1 = lax.rev(ck1, (0,))
            rv1 = lax.rev(cw1, (0,))
            m1 = bk1 <= rk1
            nk1, nv1 = plsc.sort_key_val(
                jnp.where(m1, bk1, rk1), jnp.where(m1, bv1, rv1))
            return (nk0, nv0, nk1, nv1)

        inf16 = jnp.full((16,), INF, jnp.float32)
        z16 = jnp.zeros((16,), jnp.int32)
        _, bva, _, bvb = lax.fori_loop(
            0, CHUNKS, _chunk, (inf16, z16, inf16, z16))

        _post(a0, xi0, yi0, vxi0, vyi0, bva)
        _post(a1, xi1, yi1, vxi1, vyi1, bvb)
        return carry

    lax.fori_loop(0, SC_RPW // 2, _agent_pair, 0)

    pltpu.sync_copy(outv, outh.at[pl.ds(wid * SC_RPW, SC_RPW)])


def _sc_run(x1, y1, x2, y2, W, b):
    wt = jnp.tile(W.T, (1, 2))
    bt = jnp.tile(b, 2)
    mesh = plsc.VectorSubcoreMesh(
        core_axis_name="c", subcore_axis_name="s",
        num_cores=NC, num_subcores=NS)
    kern = functools.partial(
        pl.kernel,
        out_type=jax.ShapeDtypeStruct((SC_ROWS, K * OUT_PER), jnp.float32),
        mesh=mesh,
        compiler_params=pltpu.CompilerParams(
            use_tc_tiling_on_sc=False, needs_layout_passes=False),
        scratch_types=[
            pltpu.VMEM((N,), jnp.float32),
            pltpu.VMEM((N,), jnp.float32),
            pltpu.VMEM((N,), jnp.float32),
            pltpu.VMEM((N,), jnp.float32),
            pltpu.VMEM((N,), jnp.float32),
            pltpu.VMEM((N,), jnp.float32),
            pltpu.VMEM((4, 16), jnp.float32),
            pltpu.VMEM((16,), jnp.float32),
            pltpu.VMEM((64,), jnp.float32),
            pltpu.VMEM((SC_RPW, K * OUT_PER), jnp.float32),
        ],
    )(_sc_body)
    return kern(x1, y1, x2, y2, wt, bt)


def _tc_body(x1r, y1r, x2c, y2c, x2r, y2r, wt, b2, out_ref):
    i = pl.program_id(0)
    base = i * BR

    col = lax.broadcasted_iota(jnp.int32, (BR, N), 1)
    row = base + lax.broadcasted_iota(jnp.int32, (BR, N), 0)

    relx = x2r[...] - x2c[...]
    rely = y2r[...] - y2c[...]
    dist = jnp.sqrt(relx * relx + rely * rely)
    dist = jnp.where(col == row, jnp.inf, dist)

    vxr = x2r[...] - x1r[...]
    vyr = y2r[...] - y1r[...]
    ptab = jnp.concatenate([x2r[...], y2r[...], vxr, vyr], axis=0).T

    rowhot = (col == row).astype(jnp.float32)
    self4 = jnp.dot(rowhot, ptab, preferred_element_type=jnp.float32)

    for k in range(K):
        m = jnp.min(dist, axis=1, keepdims=True)
        cand = jnp.where(dist == m, col, N)
        idx = jnp.min(cand, axis=1, keepdims=True)
        onehot = (col == idx).astype(jnp.float32)
        feats = jnp.dot(onehot, ptab, preferred_element_type=jnp.float32)
        rel = feats - self4
        emb = jnp.maximum(
            jnp.dot(rel, wt[...], preferred_element_type=jnp.float32)
            + b2[...], 0.0)
        out_ref[:, k * OUT_PER:(k + 1) * OUT_PER] = emb
        if k != K - 1:
            dist = jnp.where(col == idx, jnp.inf, dist)


def _tc_run(x1, y1, x2, y2, W, b):
    x1r = x1.reshape(1, N)
    y1r = y1.reshape(1, N)
    x2r = x2.reshape(1, N)
    y2r = y2.reshape(1, N)
    x2c = x2.reshape(N, 1)
    y2c = y2.reshape(N, 1)
    wt = W.T
    b2 = b.reshape(1, OUT_PER)

    grid = (TC_ROWS // BR,)
    full_row = pl.BlockSpec((1, N), lambda i: (0, 0))
    col_blk = pl.BlockSpec((BR, 1), lambda i: (i, 0))
    return pl.pallas_call(
        _tc_body,
        grid=grid,
        in_specs=[
            full_row, full_row,
            col_blk, col_blk,
            full_row, full_row,
            pl.BlockSpec((4, OUT_PER), lambda i: (0, 0)),
            pl.BlockSpec((1, OUT_PER), lambda i: (0, 0)),
        ],
        out_specs=pl.BlockSpec((BR, K * OUT_PER), lambda i: (i, 0)),
        out_shape=jax.ShapeDtypeStruct((TC_ROWS, K * OUT_PER), jnp.float32),
    )(x1r, y1r, x2c, y2c, x2r, y2r, wt, b2)


@jax.jit
def _run(obs1, obs2, W, b):
    x1 = obs1[:, 0]
    y1 = obs1[:, 1]
    x2 = obs2[:, 0]
    y2 = obs2[:, 1]
    sc_out = _sc_run(x1, y1, x2, y2, W, b)
    tc_out = _tc_run(x1, y1, x2, y2, W, b)
    return jnp.concatenate([tc_out, sc_out], axis=0)


def kernel(_, obs1, obs2, W, b):
    return _run(obs1, obs2, W, b)


# speedup vs baseline: 1.5182x; 1.0017x over previous
import functools

import jax
import jax.numpy as jnp
from jax import lax
from jax.experimental import pallas as pl
from jax.experimental.pallas import tpu as pltpu
from jax.experimental.pallas import tpu_sc as plsc

N = 2048
K = 8
OUT_PER = 8
BR = 256
NC = 2
NS = 16
NW = NC * NS
TC_ROWS = 1024
SC_ROWS = N - TC_ROWS
SC_RPW = SC_ROWS // NW
CHUNKS = N // 16
QI = 4
INF = float("inf")


def _sc_body(x1h, y1h, x2h, y2h, wth, bth, outh,
             x1v, y1v, x2v, y2v, vxv, vyv, wtv, btv, fbuf, outv):
    wid = lax.axis_index("s") * NC + lax.axis_index("c")
    base_row = TC_ROWS + wid * SC_RPW

    pltpu.sync_copy(x1h, x1v)
    pltpu.sync_copy(y1h, y1v)
    pltpu.sync_copy(x2h, x2v)
    pltpu.sync_copy(y2h, y2v)
    pltpu.sync_copy(wth, wtv)
    pltpu.sync_copy(bth, btv)

    io = lax.iota(jnp.int32, 16)
    khalf = lax.shift_right_logical(io, 3)

    def _vel(c, carry):
        s = pl.ds(c * 16, 16)
        vxv[s] = x2v[s] - x1v[s]
        vyv[s] = y2v[s] - y1v[s]
        return carry
    lax.fori_loop(0, CHUNKS, _vel, 0)

    btile = btv[...]
    w0 = wtv[0, :]
    w1 = wtv[1, :]
    w2 = wtv[2, :]
    w3 = wtv[3, :]

    def _post(a, xi, yi, vxi, vyi, bv):
        gx = plsc.load_gather(x2v, [bv])
        gy = plsc.load_gather(y2v, [bv])
        gvx = plsc.load_gather(vxv, [bv])
        gvy = plsc.load_gather(vyv, [bv])
        fbuf[pl.ds(0, 16)] = gx - xi
        fbuf[pl.ds(16, 16)] = gy - yi
        fbuf[pl.ds(32, 16)] = gvx - vxi
        fbuf[pl.ds(48, 16)] = gvy - vyi

        for g in range(4):
            sel = khalf + (2 * g)
            acc = btile
            acc = acc + plsc.load_gather(fbuf, [sel]) * w0
            acc = acc + plsc.load_gather(fbuf, [sel + 16]) * w1
            acc = acc + plsc.load_gather(fbuf, [sel + 32]) * w2
            acc = acc + plsc.load_gather(fbuf, [sel + 48]) * w3
            outv[a, pl.ds(g * 16, 16)] = jnp.maximum(acc, 0.0)

    def _agent_group(p, carry):
        ags = [p * QI + q for q in range(QI)]
        ivs = [jnp.full((16,), base_row + a, jnp.int32) for a in ags]
        xis = [plsc.load_gather(x2v, [iv]) for iv in ivs]
        yis = [plsc.load_gather(y2v, [iv]) for iv in ivs]
        vxis = [plsc.load_gather(vxv, [iv]) for iv in ivs]
        vyis = [plsc.load_gather(vyv, [iv]) for iv in ivs]

        def _chunk(c, bkv):
            s = pl.ds(c * 16, 16)
            civ = io + c * 16
            xs = x2v[s]
            ys = y2v[s]
            nxt = []
            for q in range(QI):
                bk, bvv = bkv[2 * q], bkv[2 * q + 1]
                dx = xs - xis[q]
                dy = ys - yis[q]
                d = dx * dx + dy * dy
                d = jnp.where(civ == ivs[q], INF, d)
                ck, cw = plsc.sort_key_val(d, civ)
                rk = lax.rev(ck, (0,))
                rv = lax.rev(cw, (0,))
                m = bk <= rk
                nk, nv = plsc.sort_key_val(
                    jnp.where(m, bk, rk), jnp.where(m, bvv, rv))
                nxt += [nk, nv]
            return tuple(nxt)

        inf16 = jnp.full((16,), INF, jnp.float32)
        z16 = jnp.zeros((16,), jnp.int32)
        res = lax.fori_loop(0, CHUNKS, _chunk, (inf16, z16) * QI)
        for q in range(QI):
            _post(ags[q], xis[q], yis[q], vxis[q], vyis[q], res[2 * q + 1])
        return carry

    lax.fori_loop(0, SC_RPW // QI, _agent_group, 0)

    pltpu.sync_copy(outv, outh.at[pl.ds(wid * SC_RPW, SC_RPW)])


def _sc_run(x1, y1, x2, y2, W, b):
    wt = jnp.tile(W.T, (1, 2))
    bt = jnp.tile(b, 2)
    mesh = plsc.VectorSubcoreMesh(
        core_axis_name="c", subcore_axis_name="s",
        num_cores=NC, num_subcores=NS)
    kern = functools.partial(
        pl.kernel,
        out_type=jax.ShapeDtypeStruct((SC_ROWS, K * OUT_PER), jnp.float32),
        mesh=mesh,
        compiler_params=pltpu.CompilerParams(
            use_tc_tiling_on_sc=False, needs_layout_passes=False),
        scratch_types=[
            pltpu.VMEM((N,), jnp.float32),
            pltpu.VMEM((N,), jnp.float32),
            pltpu.VMEM((N,), jnp.float32),
            pltpu.VMEM((N,), jnp.float32),
            pltpu.VMEM((N,), jnp.float32),
            pltpu.VMEM((N,), jnp.float32),
            pltpu.VMEM((4, 16), jnp.float32),
            pltpu.VMEM((16,), jnp.float32),
            pltpu.VMEM((64,), jnp.float32),
            pltpu.VMEM((SC_RPW, K * OUT_PER), jnp.float32),
        ],
    )(_sc_body)
    return kern(x1, y1, x2, y2, wt, bt)


def _tc_body(x1r, y1r, x2c, y2c, x2r, y2r, wt, b2, out_ref):
    i = pl.program_id(0)
    base = i * BR

    col = lax.broadcasted_iota(jnp.int32, (BR, N), 1)
    row = base + lax.broadcasted_iota(jnp.int32, (BR, N), 0)

    relx = x2r[...] - x2c[...]
    rely = y2r[...] - y2c[...]
    dist = jnp.sqrt(relx * relx + rely * rely)
    dist = jnp.where(col == row, jnp.inf, dist)

    vxr = x2r[...] - x1r[...]
    vyr = y2r[...] - y1r[...]
    ptab = jnp.concatenate([x2r[...], y2r[...], vxr, vyr], axis=0).T

    rowhot = (col == row).astype(jnp.float32)
    self4 = jnp.dot(rowhot, ptab, preferred_element_type=jnp.float32)

    for k in range(K):
        m = jnp.min(dist, axis=1, keepdims=True)
        cand = jnp.where(dist == m, col, N)
        idx = jnp.min(cand, axis=1, keepdims=True)
        onehot = (col == idx).astype(jnp.float32)
        feats = jnp.dot(onehot, ptab, preferred_element_type=jnp.float32)
        rel = feats - self4
        emb = jnp.maximum(
            jnp.dot(rel, wt[...], preferred_element_type=jnp.float32)
            + b2[...], 0.0)
        out_ref[:, k * OUT_PER:(k + 1) * OUT_PER] = emb
        if k != K - 1:
            dist = jnp.where(col == idx, jnp.inf, dist)


def _tc_run(x1, y1, x2, y2, W, b):
    x1r = x1.reshape(1, N)
    y1r = y1.reshape(1, N)
    x2r = x2.reshape(1, N)
    y2r = y2.reshape(1, N)
    x2c = x2.reshape(N, 1)
    y2c = y2.reshape(N, 1)
    wt = W.T
    b2 = b.reshape(1, OUT_PER)

    grid = (TC_ROWS // BR,)
    full_row = pl.BlockSpec((1, N), lambda i: (0, 0))
    col_blk = pl.BlockSpec((BR, 1), lambda i: (i, 0))
    return pl.pallas_call(
        _tc_body,
        grid=grid,
        in_specs=[
            full_row, full_row,
            col_blk, col_blk,
            full_row, full_row,
            pl.BlockSpec((4, OUT_PER), lambda i: (0, 0)),
            pl.BlockSpec((1, OUT_PER), lambda i: (0, 0)),
        ],
        out_specs=pl.BlockSpec((BR, K * OUT_PER), lambda i: (i, 0)),
        out_shape=jax.ShapeDtypeStruct((TC_ROWS, K * OUT_PER), jnp.float32),
    )(x1r, y1r, x2c, y2c, x2r, y2r, wt, b2)


@jax.jit
def _run(obs1, obs2, W, b):
    x1 = obs1[:, 0]
    y1 = obs1[:, 1]
    x2 = obs2[:, 0]
    y2 = obs2[:, 1]
    sc_out = _sc_run(x1, y1, x2, y2, W, b)
    tc_out = _tc_run(x1, y1, x2, y2, W, b)
    return jnp.concatenate([tc_out, sc_out], axis=0)


def kernel(_, obs1, obs2, W, b):
    return _run(obs1, obs2, W, b)


# speedup vs baseline: 1.7565x; 1.1569x over previous
import functools

import jax
import jax.numpy as jnp
from jax import lax
from jax.experimental import pallas as pl
from jax.experimental.pallas import tpu as pltpu
from jax.experimental.pallas import tpu_sc as plsc

N = 2048
K = 8
OUT_PER = 8
BR = 256
NC = 2
NS = 16
NW = NC * NS
TC_ROWS = 768
SC_ROWS = N - TC_ROWS
SC_RPW = SC_ROWS // NW
CHUNKS = N // 16
QI = 4
INF = float("inf")


def _sc_body(x1h, y1h, x2h, y2h, wth, bth, outh,
             x1v, y1v, x2v, y2v, vxv, vyv, wtv, btv, fbuf, outv):
    wid = lax.axis_index("s") * NC + lax.axis_index("c")
    base_row = TC_ROWS + wid * SC_RPW

    pltpu.sync_copy(x1h, x1v)
    pltpu.sync_copy(y1h, y1v)
    pltpu.sync_copy(x2h, x2v)
    pltpu.sync_copy(y2h, y2v)
    pltpu.sync_copy(wth, wtv)
    pltpu.sync_copy(bth, btv)

    io = lax.iota(jnp.int32, 16)
    khalf = lax.shift_right_logical(io, 3)

    def _vel(c, carry):
        s = pl.ds(c * 16, 16)
        vxv[s] = x2v[s] - x1v[s]
        vyv[s] = y2v[s] - y1v[s]
        return carry
    lax.fori_loop(0, CHUNKS, _vel, 0)

    btile = btv[...]
    w0 = wtv[0, :]
    w1 = wtv[1, :]
    w2 = wtv[2, :]
    w3 = wtv[3, :]

    def _post(a, xi, yi, vxi, vyi, bv):
        gx = plsc.load_gather(x2v, [bv])
        gy = plsc.load_gather(y2v, [bv])
        gvx = plsc.load_gather(vxv, [bv])
        gvy = plsc.load_gather(vyv, [bv])
        fbuf[pl.ds(0, 16)] = gx - xi
        fbuf[pl.ds(16, 16)] = gy - yi
        fbuf[pl.ds(32, 16)] = gvx - vxi
        fbuf[pl.ds(48, 16)] = gvy - vyi

        for g in range(4):
            sel = khalf + (2 * g)
            acc = btile
            acc = acc + plsc.load_gather(fbuf, [sel]) * w0
            acc = acc + plsc.load_gather(fbuf, [sel + 16]) * w1
            acc = acc + plsc.load_gather(fbuf, [sel + 32]) * w2
            acc = acc + plsc.load_gather(fbuf, [sel + 48]) * w3
            outv[a, pl.ds(g * 16, 16)] = jnp.maximum(acc, 0.0)

    def _agent_group(p, carry):
        ags = [p * QI + q for q in range(QI)]
        ivs = [jnp.full((16,), base_row + a, jnp.int32) for a in ags]
        xis = [plsc.load_gather(x2v, [iv]) for iv in ivs]
        yis = [plsc.load_gather(y2v, [iv]) for iv in ivs]
        vxis = [plsc.load_gather(vxv, [iv]) for iv in ivs]
        vyis = [plsc.load_gather(vyv, [iv]) for iv in ivs]

        def _chunk(c, bkv):
            s = pl.ds(c * 16, 16)
            civ = io + c * 16
            xs = x2v[s]
            ys = y2v[s]
            nxt = []
            for q in range(QI):
                bk, bvv = bkv[2 * q], bkv[2 * q + 1]
                dx = xs - xis[q]
                dy = ys - yis[q]
                d = dx * dx + dy * dy
                d = jnp.where(civ == ivs[q], INF, d)
                ck, cw = plsc.sort_key_val(d, civ)
                rk = lax.rev(ck, (0,))
                rv = lax.rev(cw, (0,))
                m = bk <= rk
                nk, nv = plsc.sort_key_val(
                    jnp.where(m, bk, rk), jnp.where(m, bvv, rv))
                nxt += [nk, nv]
            return tuple(nxt)

        inf16 = jnp.full((16,), INF, jnp.float32)
        z16 = jnp.zeros((16,), jnp.int32)
        res = lax.fori_loop(0, CHUNKS, _chunk, (inf16, z16) * QI)
        for q in range(QI):
            _post(ags[q], xis[q], yis[q], vxis[q], vyis[q], res[2 * q + 1])
        return carry

    lax.fori_loop(0, SC_RPW // QI, _agent_group, 0)

    pltpu.sync_copy(outv, outh.at[pl.ds(wid * SC_RPW, SC_RPW)])


def _sc_run(x1, y1, x2, y2, W, b):
    wt = jnp.tile(W.T, (1, 2))
    bt = jnp.tile(b, 2)
    mesh = plsc.VectorSubcoreMesh(
        core_axis_name="c", subcore_axis_name="s",
        num_cores=NC, num_subcores=NS)
    kern = functools.partial(
        pl.kernel,
        out_type=jax.ShapeDtypeStruct((SC_ROWS, K * OUT_PER), jnp.float32),
        mesh=mesh,
        compiler_params=pltpu.CompilerParams(
            use_tc_tiling_on_sc=False, needs_layout_passes=False),
        scratch_types=[
            pltpu.VMEM((N,), jnp.float32),
            pltpu.VMEM((N,), jnp.float32),
            pltpu.VMEM((N,), jnp.float32),
            pltpu.VMEM((N,), jnp.float32),
            pltpu.VMEM((N,), jnp.float32),
            pltpu.VMEM((N,), jnp.float32),
            pltpu.VMEM((4, 16), jnp.float32),
            pltpu.VMEM((16,), jnp.float32),
            pltpu.VMEM((64,), jnp.float32),
            pltpu.VMEM((SC_RPW, K * OUT_PER), jnp.float32),
        ],
    )(_sc_body)
    return kern(x1, y1, x2, y2, wt, bt)


def _tc_body(x1r, y1r, x2c, y2c, x2r, y2r, wt, b2, out_ref):
    i = pl.program_id(0)
    base = i * BR

    col = lax.broadcasted_iota(jnp.int32, (BR, N), 1)
    row = base + lax.broadcasted_iota(jnp.int32, (BR, N), 0)

    relx = x2r[...] - x2c[...]
    rely = y2r[...] - y2c[...]
    dist = jnp.sqrt(relx * relx + rely * rely)
    dist = jnp.where(col == row, jnp.inf, dist)

    vxr = x2r[...] - x1r[...]
    vyr = y2r[...] - y1r[...]
    ptab = jnp.concatenate([x2r[...], y2r[...], vxr, vyr], axis=0).T

    rowhot = (col == row).astype(jnp.float32)
    self4 = jnp.dot(rowhot, ptab, preferred_element_type=jnp.float32)

    for k in range(K):
        m = jnp.min(dist, axis=1, keepdims=True)
        cand = jnp.where(dist == m, col, N)
        idx = jnp.min(cand, axis=1, keepdims=True)
        onehot = (col == idx).astype(jnp.float32)
        feats = jnp.dot(onehot, ptab, preferred_element_type=jnp.float32)
        rel = feats - self4
        emb = jnp.maximum(
            jnp.dot(rel, wt[...], preferred_element_type=jnp.float32)
            + b2[...], 0.0)
        out_ref[:, k * OUT_PER:(k + 1) * OUT_PER] = emb
        if k != K - 1:
            dist = jnp.where(col == idx, jnp.inf, dist)


def _tc_run(x1, y1, x2, y2, W, b):
    x1r = x1.reshape(1, N)
    y1r = y1.reshape(1, N)
    x2r = x2.reshape(1, N)
    y2r = y2.reshape(1, N)
    x2c = x2.reshape(N, 1)
    y2c = y2.reshape(N, 1)
    wt = W.T
    b2 = b.reshape(1, OUT_PER)

    grid = (TC_ROWS // BR,)
    full_row = pl.BlockSpec((1, N), lambda i: (0, 0))
    col_blk = pl.BlockSpec((BR, 1), lambda i: (i, 0))
    return pl.pallas_call(
        _tc_body,
        grid=grid,
        in_specs=[
            full_row, full_row,
            col_blk, col_blk,
            full_row, full_row,
            pl.BlockSpec((4, OUT_PER), lambda i: (0, 0)),
            pl.BlockSpec((1, OUT_PER), lambda i: (0, 0)),
        ],
        out_specs=pl.BlockSpec((BR, K * OUT_PER), lambda i: (i, 0)),
        out_shape=jax.ShapeDtypeStruct((TC_ROWS, K * OUT_PER), jnp.float32),
    )(x1r, y1r, x2c, y2c, x2r, y2r, wt, b2)


@jax.jit
def _run(obs1, obs2, W, b):
    x1 = obs1[:, 0]
    y1 = obs1[:, 1]
    x2 = obs2[:, 0]
    y2 = obs2[:, 1]
    sc_out = _sc_run(x1, y1, x2, y2, W, b)
    tc_out = _tc_run(x1, y1, x2, y2, W, b)
    return jnp.concatenate([tc_out, sc_out], axis=0)


def kernel(_, obs1, obs2, W, b):
    return _run(obs1, obs2, W, b)
